# baseline (speedup 1.0000x reference)
import functools

import jax
import jax.numpy as jnp
from jax import lax
from jax.experimental import pallas as pl
from jax.experimental.pallas import tpu as pltpu
from jax.experimental.pallas import tpu_sc as plsc

_Q = 1024
_D = 16
_KB = 1024
_G = 8
_TAG_BITS = 7
_TAG_MASK = (1 << _TAG_BITS) - 1
_T = 1.00000005e-4
_NC = 2
_NS = 16


def _tc_body(w_ref, k_ref, idx_ref, kbuf, racc, *, nsteps, kb, kreal):
    j = pl.program_id(0)
    kt = k_ref[...]

    @pl.when(j < nsteps - 1)
    def _():
        kbuf[...] = jnp.concatenate([kt, kt * kt], axis=1)

    @pl.when(j == nsteps - 1)
    def _():
        row = (j * _G * kb
               + lax.broadcasted_iota(jnp.int32, (_G * kb, 1), 0))
        valid = row < kreal
        ktm = jnp.where(valid, kt, 0.0)
        ktsq = jnp.where(valid, ktm * ktm, 6.25e35)
        kbuf[...] = jnp.concatenate([ktm, ktsq], axis=1)

    sub = []
    for g in range(_G):
        sg = jnp.dot(kbuf[g * kb:(g + 1) * kb, :], w_ref[...],
                     preferred_element_type=jnp.float32)
        bg = lax.bitcast_convert_type(sg, jnp.int32)
        sub.append(lax.bitcast_convert_type(
            (bg & jnp.int32(~_TAG_MASK)) | (_G * j + g), jnp.float32))
    while len(sub) > 1:
        sub = [jnp.minimum(sub[i], sub[i + 1])
               for i in range(0, len(sub), 2)]
    tagged = sub[0]

    @pl.when(j == 0)
    def _():
        racc[...] = tagged

    @pl.when(j > 0)
    def _():
        racc[...] = jnp.minimum(racc[...], tagged)

    @pl.when(j == nsteps - 1)
    def _():
        r = racc[...]
        rmin = jnp.min(r, axis=0, keepdims=True)
        rbits = lax.bitcast_convert_type(r, jnp.int32)
        row = lax.broadcasted_iota(jnp.int32, r.shape, 0)
        gidx = (rbits & _TAG_MASK) * kb + row
        cand = jnp.where(r == rmin, gidx, jnp.int32(2**31 - 1))
        best = jnp.min(cand, axis=0, keepdims=True)
        idx_ref[...] = jnp.broadcast_to(best, (8, _Q))


def _tc_argmin(w, keys, nsteps):
    return pl.pallas_call(
        functools.partial(_tc_body, nsteps=nsteps, kb=_KB,
                          kreal=keys.shape[0]),
        grid=(nsteps,),
        in_specs=[
            pl.BlockSpec((2 * _D, _Q), lambda j: (0, 0)),
            pl.BlockSpec((_G * _KB, _D), lambda j: (j, 0)),
        ],
        out_specs=pl.BlockSpec((8, _Q), lambda j: (0, 0)),
        out_shape=jax.ShapeDtypeStruct((8, _Q), jnp.int32),
        scratch_shapes=[
            pltpu.VMEM((_G * _KB, 2 * _D), jnp.float32),
            pltpu.VMEM((_KB, _Q), jnp.float32),
        ],
    )(w, keys)


_CH = _Q // (_NC * _NS)


@functools.cache
def _make_sc_verify_gather():
    @functools.partial(
        pl.kernel,
        out_type=jax.ShapeDtypeStruct((_Q,), jnp.float32),
        mesh=plsc.VectorSubcoreMesh(core_axis_name="c", subcore_axis_name="s",
                                    num_cores=_NC, num_subcores=_NS),
        scratch_types=[
            pltpu.VMEM((_CH,), jnp.int32),
            pltpu.VMEM((_CH * _D,), jnp.int32),
            pltpu.VMEM((_CH,), jnp.float32),
            pltpu.VMEM((_CH * _D,), jnp.float32),
            pltpu.VMEM((_CH * _D,), jnp.float32),
            pltpu.VMEM((_CH,), jnp.float32),
            pltpu.SemaphoreType.DMA,
        ],
    )
    def _sc_verify_gather(idx_hbm, queries_t_hbm, keys_flat_hbm, values_hbm,
                          out_hbm, idx_v, gidx_v, val_v, qt_v, kgat_v, out_v,
                          sem):
        wid = lax.axis_index("s") * _NC + lax.axis_index("c")
        base = wid * _CH
        pltpu.sync_copy(idx_hbm.at[pl.ds(base, _CH)], idx_v)
        pltpu.async_copy(values_hbm.at[idx_v], val_v, sem).wait()
        for c in range(_D):
            pltpu.sync_copy(queries_t_hbm.at[pl.ds(c * _Q + base, _CH)],
                            qt_v.at[pl.ds(c * _CH, _CH)])
        half = [idx_v[pl.ds(0, 16)] * _D, idx_v[pl.ds(16, 16)] * _D]
        for c in range(_D):
            for h in range(_CH // 16):
                gidx_v[pl.ds(c * _CH + h * 16, 16)] = half[h] + c
        for b in range(_CH * _D // 128):
            pltpu.async_copy(
                keys_flat_hbm.at[gidx_v.at[pl.ds(b * 128, 128)]],
                kgat_v.at[pl.ds(b * 128, 128)], sem).wait()
        for t in range(_CH // 16):
            acc = jnp.zeros((16,), jnp.float32)
            for c in range(_D):
                sl = pl.ds(c * _CH + t * 16, 16)
                dv = kgat_v[sl] - qt_v[sl]
                acc = acc + dv * dv
            osl = pl.ds(t * 16, 16)
            out_v[osl] = jnp.where(acc <= _T, val_v[osl], 0.0)
        pltpu.sync_copy(out_v, out_hbm.at[pl.ds(base, _CH)])

    return _sc_verify_gather


def kernel(queries, keys, values):
    k = keys.shape[0]
    nsteps = -(-k // (_G * _KB))
    qt = queries.T
    w = jnp.concatenate([qt * (-2.0), jnp.ones((_D, _Q), jnp.float32)],
                        axis=0)
    idx = _tc_argmin(w, keys, nsteps)
    return _make_sc_verify_gather()(
        idx[0], qt.reshape(-1), keys.reshape(-1), values)


# speedup vs baseline: 1.0291x; 1.0291x over previous
import functools

import jax
import jax.numpy as jnp
from jax import lax
from jax.experimental import pallas as pl
from jax.experimental.pallas import tpu as pltpu
from jax.experimental.pallas import tpu_sc as plsc

_Q = 1024
_D = 16
_KB = 1024
_G = 16
_TAG_BITS = 7
_TAG_MASK = (1 << _TAG_BITS) - 1
_T = 1.00000005e-4
_NC = 2
_NS = 16


def _tc_body(w_ref, k_ref, idx_ref, kbuf, racc, *, nsteps, kb, kreal):
    j = pl.program_id(0)
    kt = k_ref[...]

    @pl.when(j < nsteps - 1)
    def _():
        kbuf[...] = jnp.concatenate([kt, kt * kt], axis=1)

    @pl.when(j == nsteps - 1)
    def _():
        row = (j * _G * kb
               + lax.broadcasted_iota(jnp.int32, (_G * kb, 1), 0))
        valid = row < kreal
        ktm = jnp.where(valid, kt, 0.0)
        ktsq = jnp.where(valid, ktm * ktm, 6.25e35)
        kbuf[...] = jnp.concatenate([ktm, ktsq], axis=1)

    s = jnp.dot(kbuf[...], w_ref[...],
                preferred_element_type=jnp.float32)
    sub = []
    for g in range(_G):
        sg = s[g * kb:(g + 1) * kb, :]
        bg = lax.bitcast_convert_type(sg, jnp.int32)
        sub.append(lax.bitcast_convert_type(
            (bg & jnp.int32(~_TAG_MASK)) | (_G * j + g), jnp.float32))
    while len(sub) > 1:
        sub = [jnp.minimum(sub[i], sub[i + 1])
               for i in range(0, len(sub), 2)]
    tagged = sub[0]

    @pl.when(j == 0)
    def _():
        racc[...] = tagged

    @pl.when(j > 0)
    def _():
        racc[...] = jnp.minimum(racc[...], tagged)

    @pl.when(j == nsteps - 1)
    def _():
        r = racc[...]
        rmin = jnp.min(r, axis=0, keepdims=True)
        rbits = lax.bitcast_convert_type(r, jnp.int32)
        row = lax.broadcasted_iota(jnp.int32, r.shape, 0)
        gidx = (rbits & _TAG_MASK) * kb + row
        cand = jnp.where(r == rmin, gidx, jnp.int32(2**31 - 1))
        best = jnp.min(cand, axis=0, keepdims=True)
        idx_ref[...] = jnp.broadcast_to(best, (8, _Q))


def _tc_argmin(w, keys, nsteps):
    return pl.pallas_call(
        functools.partial(_tc_body, nsteps=nsteps, kb=_KB,
                          kreal=keys.shape[0]),
        grid=(nsteps,),
        in_specs=[
            pl.BlockSpec((2 * _D, _Q), lambda j: (0, 0)),
            pl.BlockSpec((_G * _KB, _D), lambda j: (j, 0)),
        ],
        out_specs=pl.BlockSpec((8, _Q), lambda j: (0, 0)),
        out_shape=jax.ShapeDtypeStruct((8, _Q), jnp.int32),
        scratch_shapes=[
            pltpu.VMEM((_G * _KB, 2 * _D), jnp.float32),
            pltpu.VMEM((_KB, _Q), jnp.float32),
        ],
    )(w, keys)


_CH = _Q // (_NC * _NS)


@functools.cache
def _make_sc_verify_gather():
    @functools.partial(
        pl.kernel,
        out_type=jax.ShapeDtypeStruct((_Q,), jnp.float32),
        mesh=plsc.VectorSubcoreMesh(core_axis_name="c", subcore_axis_name="s",
                                    num_cores=_NC, num_subcores=_NS),
        scratch_types=[
            pltpu.VMEM((_CH,), jnp.int32),
            pltpu.VMEM((_CH * _D,), jnp.int32),
            pltpu.VMEM((_CH,), jnp.float32),
            pltpu.VMEM((_CH * _D,), jnp.float32),
            pltpu.VMEM((_CH * _D,), jnp.float32),
            pltpu.VMEM((_CH,), jnp.float32),
            pltpu.SemaphoreType.DMA,
        ],
    )
    def _sc_verify_gather(idx_hbm, queries_t_hbm, keys_flat_hbm, values_hbm,
                          out_hbm, idx_v, gidx_v, val_v, qt_v, kgat_v, out_v,
                          sem):
        wid = lax.axis_index("s") * _NC + lax.axis_index("c")
        base = wid * _CH
        pltpu.sync_copy(idx_hbm.at[pl.ds(base, _CH)], idx_v)
        pltpu.async_copy(values_hbm.at[idx_v], val_v, sem).wait()
        for c in range(_D):
            pltpu.sync_copy(queries_t_hbm.at[pl.ds(c * _Q + base, _CH)],
                            qt_v.at[pl.ds(c * _CH, _CH)])
        half = [idx_v[pl.ds(0, 16)] * _D, idx_v[pl.ds(16, 16)] * _D]
        for c in range(_D):
            for h in range(_CH // 16):
                gidx_v[pl.ds(c * _CH + h * 16, 16)] = half[h] + c
        for b in range(_CH * _D // 128):
            pltpu.async_copy(
                keys_flat_hbm.at[gidx_v.at[pl.ds(b * 128, 128)]],
                kgat_v.at[pl.ds(b * 128, 128)], sem).wait()
        for t in range(_CH // 16):
            acc = jnp.zeros((16,), jnp.float32)
            for c in range(_D):
                sl = pl.ds(c * _CH + t * 16, 16)
                dv = kgat_v[sl] - qt_v[sl]
                acc = acc + dv * dv
            osl = pl.ds(t * 16, 16)
            out_v[osl] = jnp.where(acc <= _T, val_v[osl], 0.0)
        pltpu.sync_copy(out_v, out_hbm.at[pl.ds(base, _CH)])

    return _sc_verify_gather


def kernel(queries, keys, values):
    k = keys.shape[0]
    nsteps = -(-k // (_G * _KB))
    qt = queries.T
    w = jnp.concatenate([qt * (-2.0), jnp.ones((_D, _Q), jnp.float32)],
                        axis=0)
    idx = _tc_argmin(w, keys, nsteps)
    return _make_sc_verify_gather()(
        idx[0], qt.reshape(-1), keys.reshape(-1), values)


# speedup vs baseline: 1.0831x; 1.0526x over previous
import functools

import jax
import jax.numpy as jnp
from jax import lax
from jax.experimental import pallas as pl
from jax.experimental.pallas import tpu as pltpu
from jax.experimental.pallas import tpu_sc as plsc

_Q = 1024
_D = 16
_KB = 1024
_G = 20
_TAG_BITS = 7
_TAG_MASK = (1 << _TAG_BITS) - 1
_T = 1.00000005e-4
_NC = 2
_NS = 16


def _tc_body(w_ref, k_ref, idx_ref, kbuf, racc, *, nsteps, kb, kreal):
    j = pl.program_id(0)
    kt = k_ref[...]

    @pl.when(j < nsteps - 1)
    def _():
        kbuf[...] = jnp.concatenate([kt, kt * kt], axis=1)

    @pl.when(j == nsteps - 1)
    def _():
        row = (j * _G * kb
               + lax.broadcasted_iota(jnp.int32, (_G * kb, 1), 0))
        valid = row < kreal
        ktm = jnp.where(valid, kt, 0.0)
        ktsq = jnp.where(valid, ktm * ktm, 6.25e35)
        kbuf[...] = jnp.concatenate([ktm, ktsq], axis=1)

    s = jnp.dot(kbuf[...], w_ref[...],
                preferred_element_type=jnp.float32)
    sub = []
    for g in range(_G):
        sg = s[g * kb:(g + 1) * kb, :]
        bg = lax.bitcast_convert_type(sg, jnp.int32)
        sub.append(lax.bitcast_convert_type(
            (bg & jnp.int32(~_TAG_MASK)) | (_G * j + g), jnp.float32))
    while len(sub) > 1:
        nxt = [jnp.minimum(sub[i], sub[i + 1])
               for i in range(0, len(sub) - 1, 2)]
        if len(sub) % 2:
            nxt.append(sub[-1])
        sub = nxt
    tagged = sub[0]

    @pl.when(j == 0)
    def _():
        racc[...] = tagged

    @pl.when(j > 0)
    def _():
        racc[...] = jnp.minimum(racc[...], tagged)

    @pl.when(j == nsteps - 1)
    def _():
        r = racc[...]
        rmin = jnp.min(r, axis=0, keepdims=True)
        rbits = lax.bitcast_convert_type(r, jnp.int32)
        row = lax.broadcasted_iota(jnp.int32, r.shape, 0)
        gidx = (rbits & _TAG_MASK) * kb + row
        cand = jnp.where(r == rmin, gidx, jnp.int32(2**31 - 1))
        best = jnp.min(cand, axis=0, keepdims=True)
        idx_ref[...] = jnp.broadcast_to(best, (8, _Q))


def _tc_argmin(w, keys, nsteps):
    return pl.pallas_call(
        functools.partial(_tc_body, nsteps=nsteps, kb=_KB,
                          kreal=keys.shape[0]),
        grid=(nsteps,),
        in_specs=[
            pl.BlockSpec((2 * _D, _Q), lambda j: (0, 0)),
            pl.BlockSpec((_G * _KB, _D), lambda j: (j, 0)),
        ],
        out_specs=pl.BlockSpec((8, _Q), lambda j: (0, 0)),
        out_shape=jax.ShapeDtypeStruct((8, _Q), jnp.int32),
        scratch_shapes=[
            pltpu.VMEM((_G * _KB, 2 * _D), jnp.float32),
            pltpu.VMEM((_KB, _Q), jnp.float32),
        ],
    )(w, keys)


_CH = _Q // (_NC * _NS)


@functools.cache
def _make_sc_verify_gather():
    @functools.partial(
        pl.kernel,
        out_type=jax.ShapeDtypeStruct((_Q,), jnp.float32),
        mesh=plsc.VectorSubcoreMesh(core_axis_name="c", subcore_axis_name="s",
                                    num_cores=_NC, num_subcores=_NS),
        scratch_types=[
            pltpu.VMEM((_CH,), jnp.int32),
            pltpu.VMEM((_CH * _D,), jnp.int32),
            pltpu.VMEM((_CH,), jnp.float32),
            pltpu.VMEM((_CH * _D,), jnp.float32),
            pltpu.VMEM((_CH * _D,), jnp.float32),
            pltpu.VMEM((_CH,), jnp.float32),
            pltpu.SemaphoreType.DMA,
        ],
    )
    def _sc_verify_gather(idx_hbm, queries_t_hbm, keys_flat_hbm, values_hbm,
                          out_hbm, idx_v, gidx_v, val_v, qt_v, kgat_v, out_v,
                          sem):
        wid = lax.axis_index("s") * _NC + lax.axis_index("c")
        base = wid * _CH
        pltpu.sync_copy(idx_hbm.at[pl.ds(base, _CH)], idx_v)
        pltpu.async_copy(values_hbm.at[idx_v], val_v, sem).wait()
        for c in range(_D):
            pltpu.sync_copy(queries_t_hbm.at[pl.ds(c * _Q + base, _CH)],
                            qt_v.at[pl.ds(c * _CH, _CH)])
        half = [idx_v[pl.ds(0, 16)] * _D, idx_v[pl.ds(16, 16)] * _D]
        for c in range(_D):
            for h in range(_CH // 16):
                gidx_v[pl.ds(c * _CH + h * 16, 16)] = half[h] + c
        for b in range(_CH * _D // 128):
            pltpu.async_copy(
                keys_flat_hbm.at[gidx_v.at[pl.ds(b * 128, 128)]],
                kgat_v.at[pl.ds(b * 128, 128)], sem).wait()
        for t in range(_CH // 16):
            acc = jnp.zeros((16,), jnp.float32)
            for c in range(_D):
                sl = pl.ds(c * _CH + t * 16, 16)
                dv = kgat_v[sl] - qt_v[sl]
                acc = acc + dv * dv
            osl = pl.ds(t * 16, 16)
            out_v[osl] = jnp.where(acc <= _T, val_v[osl], 0.0)
        pltpu.sync_copy(out_v, out_hbm.at[pl.ds(base, _CH)])

    return _sc_verify_gather


def kernel(queries, keys, values):
    k = keys.shape[0]
    nsteps = -(-k // (_G * _KB))
    qt = queries.T
    w = jnp.concatenate([qt * (-2.0), jnp.ones((_D, _Q), jnp.float32)],
                        axis=0)
    idx = _tc_argmin(w, keys, nsteps)
    return _make_sc_verify_gather()(
        idx[0], qt.reshape(-1), keys.reshape(-1), values)
